# Initial kernel scaffold; baseline (speedup 1.0000x reference)
import jax
import jax.numpy as jnp
from jax.experimental import pallas as pl


def kernel(x, edge_index, edge_attr, pos, Wq1, Wk1, Wv1, Wq2, Wk2, Wv2):
    raise NotImplementedError("write your pallas kernel here")


# speedup vs baseline: 42.9142x; 42.9142x over previous
import functools

import jax
import jax.numpy as jnp
import numpy as np
from jax import lax
from jax.experimental import pallas as pl
from jax.experimental.pallas import tpu as pltpu
from jax.experimental.pallas import tpu_sc as plsc

N = 50000
E = 800000
D_IN = 15
D_EDGE = 5
NUM_RBF = 8
HID = 32
HEADS = 4
HEAD_DIM = HID // HEADS
CUTOFF = 8.0

NUM_CORES = 2
NUM_SUBCORES = 16
NW = NUM_CORES * NUM_SUBCORES
EPT = E // NW
MAC = 500
NMAC = EPT // MAC
SUB = 125
NSUB = MAC // SUB
AUG = 48
GW = 128
ACC_W = 48

HN = N // 2
EPT2 = E // NUM_SUBCORES
NMAC2 = EPT2 // MAC
ZR2 = 3126
OR2 = HN // 8

_GAMMA = CUTOFF / NUM_RBF
_SCALE = 1.0 / np.sqrt(float(HEAD_DIM))


@functools.cache
def _mesh():
  return plsc.VectorSubcoreMesh(
      core_axis_name="c", subcore_axis_name="s",
      num_cores=NUM_CORES, num_subcores=NUM_SUBCORES,
  )


def _centers_row():
  i = lax.broadcasted_iota(jnp.int32, (1, NUM_RBF), 1)
  return i.astype(jnp.float32) * (CUTOFF / (NUM_RBF - 1))


def _sel():
  r = lax.broadcasted_iota(jnp.int32, (HID, HEADS), 0) // HEAD_DIM
  c = lax.broadcasted_iota(jnp.int32, (HID, HEADS), 1)
  return (r == c).astype(jnp.float32)


def _selt():
  r = lax.broadcasted_iota(jnp.int32, (HEADS, HID), 0)
  c = lax.broadcasted_iota(jnp.int32, (HEADS, HID), 1) // HEAD_DIM
  return (r == c).astype(jnp.float32)


def _gather_call(src2d, dst2d, tq, tk, tv):
  widths = (AUG, AUG, HID)
  offs = (0, AUG, 2 * AUG)
  use_dst = (True, False, False)

  @functools.partial(
      pl.kernel,
      out_type=jax.ShapeDtypeStruct((E, GW), jnp.float32),
      mesh=_mesh(),
      compiler_params=pltpu.CompilerParams(use_tc_tiling_on_sc=False),
      scratch_types=[
          pltpu.VMEM((NSUB, SUB), jnp.int32),
          pltpu.VMEM((NSUB, SUB), jnp.int32),
          pltpu.VMEM((MAC, AUG), jnp.float32),
          pltpu.VMEM((MAC, AUG), jnp.float32),
          pltpu.VMEM((MAC, HID), jnp.float32),
          pltpu.SemaphoreType.DMA,
      ],
  )
  def gather_kernel(src2d_h, dst2d_h, tq_h, tk_h, tv_h, g_out,
                    idx_src, idx_dst, bq, bk, bv, sem):
    c = lax.axis_index("c")
    s = lax.axis_index("s")
    wid = s * NUM_CORES + c
    tables = (tq_h, tk_h, tv_h)
    bufs = (bq, bk, bv)

    def body(m, carry):
      base = wid * EPT + m * MAC
      r0 = wid * (EPT // SUB) + m * NSUB
      pltpu.sync_copy(src2d_h.at[pl.ds(r0, NSUB)], idx_src)
      pltpu.sync_copy(dst2d_h.at[pl.ds(r0, NSUB)], idx_dst)
      copies = []
      for j in range(NSUB):
        for tab, buf, dflag in zip(tables, bufs, use_dst):
          idx = (idx_dst if dflag else idx_src).at[j]
          copies.append(pltpu.async_copy(
              tab.at[idx], buf.at[pl.ds(j * SUB, SUB)], sem))
      for cp in copies:
        cp.wait()
      for buf, off, w in zip(bufs, offs, widths):
        pltpu.sync_copy(buf, g_out.at[pl.ds(base, MAC), pl.ds(off, w)])
      return carry

    lax.fori_loop(0, NMAC, body, 0)

  return gather_kernel(src2d, dst2d, tq, tk, tv)


def _scatter_call(dstc2d, rows, zeros):
  @functools.partial(
      pl.kernel,
      out_type=jax.ShapeDtypeStruct((N, ACC_W), jnp.float32),
      mesh=_mesh(),
      compiler_params=pltpu.CompilerParams(use_tc_tiling_on_sc=False),
      scratch_types=[
          pltpu.VMEM((NSUB, SUB), jnp.int32),
          pltpu.VMEM((MAC, ACC_W), jnp.float32),
          pltpu.VMEM_SHARED((HN + 8, ACC_W), jnp.float32),
          pltpu.SemaphoreType.DMA,
      ],
  )
  def scatter_kernel(dstc2d_h, rows_h, zeros_h, out, idx_v, rows_v, acc, sem):
    c = lax.axis_index("c")
    s = lax.axis_index("s")

    @pl.when(s < 8)
    def _zero():
      pltpu.sync_copy(zeros_h, acc.at[pl.ds(s * ZR2, ZR2)])

    plsc.subcore_barrier()

    def body(m, carry):
      base = s * EPT2 + m * MAC
      r0 = s * (EPT2 // SUB) + m * NSUB
      pltpu.sync_copy(dstc2d_h.at[c, pl.ds(r0, NSUB)], idx_v)
      pltpu.sync_copy(rows_h.at[pl.ds(base, MAC), pl.ds(0, ACC_W)], rows_v)
      copies = []
      for j in range(NSUB):
        copies.append(pltpu.async_copy(
            rows_v.at[pl.ds(j * SUB, SUB)], acc.at[idx_v.at[j]], sem,
            add=True))
      for cp in copies:
        cp.wait()
      return carry

    lax.fori_loop(0, NMAC2, body, 0)
    plsc.subcore_barrier()

    @pl.when(s < 8)
    def _out():
      pltpu.sync_copy(acc.at[pl.ds(s * OR2, OR2)],
                      out.at[pl.ds(c * HN + s * OR2, OR2)])

  return scatter_kernel(dstc2d, rows, zeros)


def _dstidx_call(dst):
  rows = E // 128

  def body(dst_ref, out_ref):
    dv = dst_ref[...]
    in0 = dv < HN
    out_ref[0] = jnp.where(in0, dv, HN)
    out_ref[1] = jnp.where(in0, HN, dv - HN)

  return pl.pallas_call(
      body,
      grid=(1,),
      in_specs=[pl.BlockSpec((rows, 128), lambda i: (0, 0))],
      out_specs=pl.BlockSpec((NUM_CORES, rows, 128), lambda i: (0, 0, 0)),
      out_shape=jax.ShapeDtypeStruct((NUM_CORES, rows, 128), jnp.int32),
  )(dst.reshape(rows, 128))


def _prep_call(x, pos, Wq, Wkt, Wvt):
  d_in = x.shape[1]
  bn = 5000

  def body(x_ref, pos_ref, wq, wk, wv, q_ref, k_ref, v_ref):
    xb = x_ref[...]
    pb = pos_ref[...]
    pad = jnp.zeros((bn, AUG - HID - 3), jnp.float32)
    q = jnp.dot(xb, wq[...], preferred_element_type=jnp.float32)
    k = jnp.dot(xb, wk[...], preferred_element_type=jnp.float32)
    q_ref[...] = jnp.concatenate([q, pb, pad], axis=1)
    k_ref[...] = jnp.concatenate([k, pb, pad], axis=1)
    v_ref[...] = jnp.dot(xb, wv[...], preferred_element_type=jnp.float32)

  return pl.pallas_call(
      body,
      grid=(N // bn,),
      in_specs=[
          pl.BlockSpec((bn, d_in), lambda i: (i, 0)),
          pl.BlockSpec((bn, 3), lambda i: (i, 0)),
          pl.BlockSpec((d_in, HID), lambda i: (0, 0)),
          pl.BlockSpec((d_in, HID), lambda i: (0, 0)),
          pl.BlockSpec((d_in, HID), lambda i: (0, 0)),
      ],
      out_specs=[
          pl.BlockSpec((bn, AUG), lambda i: (i, 0)),
          pl.BlockSpec((bn, AUG), lambda i: (i, 0)),
          pl.BlockSpec((bn, HID), lambda i: (i, 0)),
      ],
      out_shape=[
          jax.ShapeDtypeStruct((N, AUG), jnp.float32),
          jax.ShapeDtypeStruct((N, AUG), jnp.float32),
          jax.ShapeDtypeStruct((N, HID), jnp.float32),
      ],
  )(x, pos, Wq, Wkt, Wvt)


def _edge_call(g, eat, Wka, Wkr, Wva, Wvr):
  be = 6400

  def body(g_ref, eat_ref, wka, wkr, wva, wvr, rows_ref):
    gb = g_ref[...]
    q = gb[:, 0:HID]
    posd = gb[:, HID:HID + 4]
    k0 = gb[:, AUG:AUG + HID]
    poss = gb[:, AUG + HID:AUG + HID + 4]
    v0 = gb[:, 2 * AUG:2 * AUG + HID]
    diff = posd - poss
    dist = jnp.sqrt(jnp.sum(diff * diff, axis=1, keepdims=True) + 1e-9)
    rbf = jnp.exp(-((dist - _centers_row()) ** 2) / (_GAMMA ** 2))
    ea_k = lax.dot_general(eat_ref[...], wka[...], (((0,), (0,)), ((), ())),
                           preferred_element_type=jnp.float32)
    ea_v = lax.dot_general(eat_ref[...], wva[...], (((0,), (0,)), ((), ())),
                           preferred_element_type=jnp.float32)
    k = (k0 + ea_k
         + jnp.dot(rbf, wkr[...], preferred_element_type=jnp.float32))
    v = (v0 + ea_v
         + jnp.dot(rbf, wvr[...], preferred_element_type=jnp.float32))
    t = q * k
    logits = jnp.dot(t, _sel(), preferred_element_type=jnp.float32) * _SCALE
    p = jnp.exp(logits)
    pfull = jnp.dot(p, _selt(), preferred_element_type=jnp.float32)
    rows_ref[...] = jnp.concatenate(
        [p, jnp.zeros((be, 4), jnp.float32), pfull * v,
         jnp.zeros((be, GW - 40), jnp.float32)], axis=1)

  return pl.pallas_call(
      body,
      grid=(E // be,),
      in_specs=[
          pl.BlockSpec((be, GW), lambda i: (i, 0)),
          pl.BlockSpec((D_EDGE, be), lambda i: (0, i)),
          pl.BlockSpec((D_EDGE, HID), lambda i: (0, 0)),
          pl.BlockSpec((NUM_RBF, HID), lambda i: (0, 0)),
          pl.BlockSpec((D_EDGE, HID), lambda i: (0, 0)),
          pl.BlockSpec((NUM_RBF, HID), lambda i: (0, 0)),
      ],
      out_specs=pl.BlockSpec((be, GW), lambda i: (i, 0)),
      out_shape=jax.ShapeDtypeStruct((E, GW), jnp.float32),
  )(g, eat, Wka, Wkr, Wva, Wvr)


def _combine1_call(part, pos, Wq, Wkt, Wvt):
  bn = 5000

  def body(part_ref, pos_ref, wq, wk, wv, q_ref, k_ref, v_ref):
    a = part_ref[...]
    den = jnp.dot(a[:, 0:4], _selt(), preferred_element_type=jnp.float32) + 1e-9
    h = jnp.maximum(a[:, 8:40] / den, 0.0)
    pb = pos_ref[...]
    pad = jnp.zeros((bn, AUG - HID - 3), jnp.float32)
    q = jnp.dot(h, wq[...], preferred_element_type=jnp.float32)
    k = jnp.dot(h, wk[...], preferred_element_type=jnp.float32)
    q_ref[...] = jnp.concatenate([q, pb, pad], axis=1)
    k_ref[...] = jnp.concatenate([k, pb, pad], axis=1)
    v_ref[...] = jnp.dot(h, wv[...], preferred_element_type=jnp.float32)

  return pl.pallas_call(
      body,
      grid=(N // bn,),
      in_specs=[
          pl.BlockSpec((bn, ACC_W), lambda i: (i, 0)),
          pl.BlockSpec((bn, 3), lambda i: (i, 0)),
          pl.BlockSpec((HID, HID), lambda i: (0, 0)),
          pl.BlockSpec((HID, HID), lambda i: (0, 0)),
          pl.BlockSpec((HID, HID), lambda i: (0, 0)),
      ],
      out_specs=[
          pl.BlockSpec((bn, AUG), lambda i: (i, 0)),
          pl.BlockSpec((bn, AUG), lambda i: (i, 0)),
          pl.BlockSpec((bn, HID), lambda i: (i, 0)),
      ],
      out_shape=[
          jax.ShapeDtypeStruct((N, AUG), jnp.float32),
          jax.ShapeDtypeStruct((N, AUG), jnp.float32),
          jax.ShapeDtypeStruct((N, HID), jnp.float32),
      ],
  )(part, pos, Wq, Wkt, Wvt)


def _combine2_call(part):
  bn = 5000

  def body(part_ref, h_ref):
    a = part_ref[...]
    den = jnp.dot(a[:, 0:4], _selt(), preferred_element_type=jnp.float32) + 1e-9
    h_ref[...] = a[:, 8:40] / den

  return pl.pallas_call(
      body,
      grid=(N // bn,),
      in_specs=[pl.BlockSpec((bn, ACC_W), lambda i: (i, 0))],
      out_specs=pl.BlockSpec((bn, HID), lambda i: (i, 0)),
      out_shape=jax.ShapeDtypeStruct((N, HID), jnp.float32),
  )(part)


def kernel(x, edge_index, edge_attr, pos, Wq1, Wk1, Wv1, Wq2, Wk2, Wv2):
  src2d = edge_index[0].reshape(E // SUB, SUB)
  dst2d = edge_index[1].reshape(E // SUB, SUB)
  eat = edge_attr.T

  Wk1t, Wk1a, Wk1r = Wk1[:D_IN], Wk1[D_IN:D_IN + D_EDGE], Wk1[D_IN + D_EDGE:]
  Wv1t, Wv1a, Wv1r = Wv1[:D_IN], Wv1[D_IN:D_IN + D_EDGE], Wv1[D_IN + D_EDGE:]
  Wk2t, Wk2a, Wk2r = Wk2[:HID], Wk2[HID:HID + D_EDGE], Wk2[HID + D_EDGE:]
  Wv2t, Wv2a, Wv2r = Wv2[:HID], Wv2[HID:HID + D_EDGE], Wv2[HID + D_EDGE:]

  zeros = jnp.zeros((ZR2, ACC_W), jnp.float32)
  dstc2d = _dstidx_call(edge_index[1]).reshape(NUM_CORES, E // SUB, SUB)

  tq1, tk1, tv1 = _prep_call(x, pos, Wq1, Wk1t, Wv1t)
  g1 = _gather_call(src2d, dst2d, tq1, tk1, tv1)
  rows1 = _edge_call(g1, eat, Wk1a, Wk1r, Wv1a, Wv1r)
  part1 = _scatter_call(dstc2d, rows1, zeros)

  tq2, tk2, tv2 = _combine1_call(part1, pos, Wq2, Wk2t, Wv2t)
  g2 = _gather_call(src2d, dst2d, tq2, tk2, tv2)
  rows2 = _edge_call(g2, eat, Wk2a, Wk2r, Wv2a, Wv2r)
  part2 = _scatter_call(dstc2d, rows2, zeros)
  return _combine2_call(part2)


# speedup vs baseline: 45.6828x; 1.0645x over previous
import functools

import jax
import jax.numpy as jnp
import numpy as np
from jax import lax
from jax.experimental import pallas as pl
from jax.experimental.pallas import tpu as pltpu
from jax.experimental.pallas import tpu_sc as plsc

N = 50000
E = 800000
D_IN = 15
D_EDGE = 5
NUM_RBF = 8
HID = 32
HEADS = 4
HEAD_DIM = HID // HEADS
CUTOFF = 8.0

NUM_CORES = 2
NUM_SUBCORES = 16
NW = NUM_CORES * NUM_SUBCORES
EPT = E // NW
MAC = 500
NMAC = EPT // MAC
SUB = 125
NSUB = MAC // SUB
AUG = 48
GW = 128
ACC_W = 48

HN = N // 2
NTRASH = 128
ER = E // 2
EPR = ER // NUM_SUBCORES
NMAC2 = EPR // MAC
ZR2 = (HN + NTRASH) // 8
OR2 = HN // 8

_GAMMA = CUTOFF / NUM_RBF
_SCALE = 1.0 / np.sqrt(float(HEAD_DIM))


@functools.cache
def _mesh():
  return plsc.VectorSubcoreMesh(
      core_axis_name="c", subcore_axis_name="s",
      num_cores=NUM_CORES, num_subcores=NUM_SUBCORES,
  )


def _centers_row():
  i = lax.broadcasted_iota(jnp.int32, (1, NUM_RBF), 1)
  return i.astype(jnp.float32) * (CUTOFF / (NUM_RBF - 1))


def _sel():
  r = lax.broadcasted_iota(jnp.int32, (HID, HEADS), 0) // HEAD_DIM
  c = lax.broadcasted_iota(jnp.int32, (HID, HEADS), 1)
  return (r == c).astype(jnp.float32)


def _selt():
  r = lax.broadcasted_iota(jnp.int32, (HEADS, HID), 0)
  c = lax.broadcasted_iota(jnp.int32, (HEADS, HID), 1) // HEAD_DIM
  return (r == c).astype(jnp.float32)


def _gather_call(src2d, dst2d, tq, tk, tv):
  widths = (AUG, AUG, HID)
  offs = (0, AUG, 2 * AUG)
  use_dst = (True, False, False)

  @functools.partial(
      pl.kernel,
      out_type=jax.ShapeDtypeStruct((E, GW), jnp.float32),
      mesh=_mesh(),
      compiler_params=pltpu.CompilerParams(use_tc_tiling_on_sc=False),
      scratch_types=[
          pltpu.VMEM((NSUB, SUB), jnp.int32),
          pltpu.VMEM((NSUB, SUB), jnp.int32),
          pltpu.VMEM((MAC, AUG), jnp.float32),
          pltpu.VMEM((MAC, AUG), jnp.float32),
          pltpu.VMEM((MAC, HID), jnp.float32),
          pltpu.SemaphoreType.DMA,
      ],
  )
  def gather_kernel(src2d_h, dst2d_h, tq_h, tk_h, tv_h, g_out,
                    idx_src, idx_dst, bq, bk, bv, sem):
    c = lax.axis_index("c")
    s = lax.axis_index("s")
    wid = s * NUM_CORES + c
    tables = (tq_h, tk_h, tv_h)
    bufs = (bq, bk, bv)

    def body(m, carry):
      base = wid * EPT + m * MAC
      r0 = wid * (EPT // SUB) + m * NSUB
      pltpu.sync_copy(src2d_h.at[pl.ds(r0, NSUB)], idx_src)
      pltpu.sync_copy(dst2d_h.at[pl.ds(r0, NSUB)], idx_dst)
      copies = []
      for j in range(NSUB):
        for tab, buf, dflag in zip(tables, bufs, use_dst):
          idx = (idx_dst if dflag else idx_src).at[j]
          copies.append(pltpu.async_copy(
              tab.at[idx], buf.at[pl.ds(j * SUB, SUB)], sem))
      for cp in copies:
        cp.wait()
      for buf, off, w in zip(bufs, offs, widths):
        pltpu.sync_copy(buf, g_out.at[pl.ds(base, MAC), pl.ds(off, w)])
      return carry

    lax.fori_loop(0, NMAC, body, 0)

  return gather_kernel(src2d, dst2d, tq, tk, tv)


def _scatter_call(dstc2d, rows, zeros):
  @functools.partial(
      pl.kernel,
      out_type=jax.ShapeDtypeStruct((N, ACC_W), jnp.float32),
      mesh=_mesh(),
      compiler_params=pltpu.CompilerParams(use_tc_tiling_on_sc=False),
      scratch_types=[
          pltpu.VMEM((NSUB, SUB), jnp.int32),
          pltpu.VMEM((NSUB, SUB), jnp.int32),
          pltpu.VMEM((MAC, ACC_W), jnp.float32),
          pltpu.VMEM((MAC, ACC_W), jnp.float32),
          pltpu.VMEM_SHARED((HN + NTRASH, ACC_W), jnp.float32),
          pltpu.SemaphoreType.DMA,
      ],
  )
  def scatter_kernel(dstc2d_h, rows_h, zeros_h, out, idx_a, idx_b,
                     rows_a, rows_b, acc, sem):
    c = lax.axis_index("c")
    s = lax.axis_index("s")

    @pl.when(s < 8)
    def _zero():
      pltpu.sync_copy(zeros_h, acc.at[pl.ds(s * ZR2, ZR2)])

    plsc.subcore_barrier()

    def body(m, carry):
      base = s * EPR + m * MAC
      r0 = s * (EPR // SUB) + m * NSUB
      pltpu.sync_copy(dstc2d_h.at[c, pl.ds(r0, NSUB)], idx_a)
      pltpu.sync_copy(dstc2d_h.at[c, pl.ds(r0 + ER // SUB, NSUB)], idx_b)
      pltpu.sync_copy(rows_h.at[pl.ds(base, MAC), pl.ds(0, ACC_W)], rows_a)
      pltpu.sync_copy(rows_h.at[pl.ds(base, MAC), pl.ds(ACC_W, ACC_W)],
                      rows_b)
      copies = []
      for j in range(NSUB):
        copies.append(pltpu.async_copy(
            rows_a.at[pl.ds(j * SUB, SUB)], acc.at[idx_a.at[j]], sem,
            add=True))
        copies.append(pltpu.async_copy(
            rows_b.at[pl.ds(j * SUB, SUB)], acc.at[idx_b.at[j]], sem,
            add=True))
      for cp in copies:
        cp.wait()
      return carry

    lax.fori_loop(0, NMAC2, body, 0)
    plsc.subcore_barrier()

    @pl.when(s < 8)
    def _out():
      pltpu.sync_copy(acc.at[pl.ds(s * OR2, OR2)],
                      out.at[pl.ds(c * HN + s * OR2, OR2)])

  return scatter_kernel(dstc2d, rows, zeros)


def _dstidx_call(dst):
  rows = E // 128

  def body(dst_ref, out_ref):
    dv = dst_ref[...]
    in0 = dv < HN
    trash = HN + (lax.broadcasted_iota(jnp.int32, dv.shape, 1)
                  % jnp.int32(NTRASH))
    out_ref[0] = jnp.where(in0, dv, trash)
    out_ref[1] = jnp.where(in0, trash, dv - HN)

  return pl.pallas_call(
      body,
      grid=(1,),
      in_specs=[pl.BlockSpec((rows, 128), lambda i: (0, 0))],
      out_specs=pl.BlockSpec((NUM_CORES, rows, 128), lambda i: (0, 0, 0)),
      out_shape=jax.ShapeDtypeStruct((NUM_CORES, rows, 128), jnp.int32),
  )(dst.reshape(rows, 128))


def _prep_call(x, pos, Wq, Wkt, Wvt):
  d_in = x.shape[1]
  bn = 5000

  def body(x_ref, pos_ref, wq, wk, wv, q_ref, k_ref, v_ref):
    xb = x_ref[...]
    pb = pos_ref[...]
    pad = jnp.zeros((bn, AUG - HID - 3), jnp.float32)
    q = jnp.dot(xb, wq[...], preferred_element_type=jnp.float32)
    k = jnp.dot(xb, wk[...], preferred_element_type=jnp.float32)
    q_ref[...] = jnp.concatenate([q, pb, pad], axis=1)
    k_ref[...] = jnp.concatenate([k, pb, pad], axis=1)
    v_ref[...] = jnp.dot(xb, wv[...], preferred_element_type=jnp.float32)

  return pl.pallas_call(
      body,
      grid=(N // bn,),
      in_specs=[
          pl.BlockSpec((bn, d_in), lambda i: (i, 0)),
          pl.BlockSpec((bn, 3), lambda i: (i, 0)),
          pl.BlockSpec((d_in, HID), lambda i: (0, 0)),
          pl.BlockSpec((d_in, HID), lambda i: (0, 0)),
          pl.BlockSpec((d_in, HID), lambda i: (0, 0)),
      ],
      out_specs=[
          pl.BlockSpec((bn, AUG), lambda i: (i, 0)),
          pl.BlockSpec((bn, AUG), lambda i: (i, 0)),
          pl.BlockSpec((bn, HID), lambda i: (i, 0)),
      ],
      out_shape=[
          jax.ShapeDtypeStruct((N, AUG), jnp.float32),
          jax.ShapeDtypeStruct((N, AUG), jnp.float32),
          jax.ShapeDtypeStruct((N, HID), jnp.float32),
      ],
  )(x, pos, Wq, Wkt, Wvt)


def _edge_rows48(gb, eatb, wka, wkr, wva, wvr):
  be = gb.shape[0]
  q = gb[:, 0:HID]
  posd = gb[:, HID:HID + 4]
  k0 = gb[:, AUG:AUG + HID]
  poss = gb[:, AUG + HID:AUG + HID + 4]
  v0 = gb[:, 2 * AUG:2 * AUG + HID]
  diff = posd - poss
  dist = jnp.sqrt(jnp.sum(diff * diff, axis=1, keepdims=True) + 1e-9)
  rbf = jnp.exp(-((dist - _centers_row()) ** 2) / (_GAMMA ** 2))
  ea_k = lax.dot_general(eatb, wka, (((0,), (0,)), ((), ())),
                         preferred_element_type=jnp.float32)
  ea_v = lax.dot_general(eatb, wva, (((0,), (0,)), ((), ())),
                         preferred_element_type=jnp.float32)
  k = k0 + ea_k + jnp.dot(rbf, wkr, preferred_element_type=jnp.float32)
  v = v0 + ea_v + jnp.dot(rbf, wvr, preferred_element_type=jnp.float32)
  t = q * k
  logits = jnp.dot(t, _sel(), preferred_element_type=jnp.float32) * _SCALE
  p = jnp.exp(logits)
  pfull = jnp.dot(p, _selt(), preferred_element_type=jnp.float32)
  return jnp.concatenate(
      [p, jnp.zeros((be, 4), jnp.float32), pfull * v,
       jnp.zeros((be, ACC_W - 40), jnp.float32)], axis=1)


def _edge_call(g, eat, Wka, Wkr, Wva, Wvr):
  be = 3200
  nblk = ER // be

  def body(ga_ref, gb_ref, eat_a, eat_b, wka, wkr, wva, wvr, rows_ref):
    ra = _edge_rows48(ga_ref[...], eat_a[...], wka[...], wkr[...],
                      wva[...], wvr[...])
    rb = _edge_rows48(gb_ref[...], eat_b[...], wka[...], wkr[...],
                      wva[...], wvr[...])
    rows_ref[...] = jnp.concatenate(
        [ra, rb, jnp.zeros((be, GW - 2 * ACC_W), jnp.float32)], axis=1)

  return pl.pallas_call(
      body,
      grid=(nblk,),
      in_specs=[
          pl.BlockSpec((be, GW), lambda i: (i, 0)),
          pl.BlockSpec((be, GW), lambda i: (i + nblk, 0)),
          pl.BlockSpec((D_EDGE, be), lambda i: (0, i)),
          pl.BlockSpec((D_EDGE, be), lambda i: (0, i + nblk)),
          pl.BlockSpec((D_EDGE, HID), lambda i: (0, 0)),
          pl.BlockSpec((NUM_RBF, HID), lambda i: (0, 0)),
          pl.BlockSpec((D_EDGE, HID), lambda i: (0, 0)),
          pl.BlockSpec((NUM_RBF, HID), lambda i: (0, 0)),
      ],
      out_specs=pl.BlockSpec((be, GW), lambda i: (i, 0)),
      out_shape=jax.ShapeDtypeStruct((ER, GW), jnp.float32),
  )(g, g, eat, eat, Wka, Wkr, Wva, Wvr)


def _combine1_call(part, pos, Wq, Wkt, Wvt):
  bn = 5000

  def body(part_ref, pos_ref, wq, wk, wv, q_ref, k_ref, v_ref):
    a = part_ref[...]
    den = jnp.dot(a[:, 0:4], _selt(), preferred_element_type=jnp.float32) + 1e-9
    h = jnp.maximum(a[:, 8:40] / den, 0.0)
    pb = pos_ref[...]
    pad = jnp.zeros((bn, AUG - HID - 3), jnp.float32)
    q = jnp.dot(h, wq[...], preferred_element_type=jnp.float32)
    k = jnp.dot(h, wk[...], preferred_element_type=jnp.float32)
    q_ref[...] = jnp.concatenate([q, pb, pad], axis=1)
    k_ref[...] = jnp.concatenate([k, pb, pad], axis=1)
    v_ref[...] = jnp.dot(h, wv[...], preferred_element_type=jnp.float32)

  return pl.pallas_call(
      body,
      grid=(N // bn,),
      in_specs=[
          pl.BlockSpec((bn, ACC_W), lambda i: (i, 0)),
          pl.BlockSpec((bn, 3), lambda i: (i, 0)),
          pl.BlockSpec((HID, HID), lambda i: (0, 0)),
          pl.BlockSpec((HID, HID), lambda i: (0, 0)),
          pl.BlockSpec((HID, HID), lambda i: (0, 0)),
      ],
      out_specs=[
          pl.BlockSpec((bn, AUG), lambda i: (i, 0)),
          pl.BlockSpec((bn, AUG), lambda i: (i, 0)),
          pl.BlockSpec((bn, HID), lambda i: (i, 0)),
      ],
      out_shape=[
          jax.ShapeDtypeStruct((N, AUG), jnp.float32),
          jax.ShapeDtypeStruct((N, AUG), jnp.float32),
          jax.ShapeDtypeStruct((N, HID), jnp.float32),
      ],
  )(part, pos, Wq, Wkt, Wvt)


def _combine2_call(part):
  bn = 5000

  def body(part_ref, h_ref):
    a = part_ref[...]
    den = jnp.dot(a[:, 0:4], _selt(), preferred_element_type=jnp.float32) + 1e-9
    h_ref[...] = a[:, 8:40] / den

  return pl.pallas_call(
      body,
      grid=(N // bn,),
      in_specs=[pl.BlockSpec((bn, ACC_W), lambda i: (i, 0))],
      out_specs=pl.BlockSpec((bn, HID), lambda i: (i, 0)),
      out_shape=jax.ShapeDtypeStruct((N, HID), jnp.float32),
  )(part)


def kernel(x, edge_index, edge_attr, pos, Wq1, Wk1, Wv1, Wq2, Wk2, Wv2):
  src2d = edge_index[0].reshape(E // SUB, SUB)
  dst2d = edge_index[1].reshape(E // SUB, SUB)
  eat = edge_attr.T

  Wk1t, Wk1a, Wk1r = Wk1[:D_IN], Wk1[D_IN:D_IN + D_EDGE], Wk1[D_IN + D_EDGE:]
  Wv1t, Wv1a, Wv1r = Wv1[:D_IN], Wv1[D_IN:D_IN + D_EDGE], Wv1[D_IN + D_EDGE:]
  Wk2t, Wk2a, Wk2r = Wk2[:HID], Wk2[HID:HID + D_EDGE], Wk2[HID + D_EDGE:]
  Wv2t, Wv2a, Wv2r = Wv2[:HID], Wv2[HID:HID + D_EDGE], Wv2[HID + D_EDGE:]

  zeros = jnp.zeros((ZR2, ACC_W), jnp.float32)
  dstc2d = _dstidx_call(edge_index[1]).reshape(NUM_CORES, E // SUB, SUB)

  tq1, tk1, tv1 = _prep_call(x, pos, Wq1, Wk1t, Wv1t)
  g1 = _gather_call(src2d, dst2d, tq1, tk1, tv1)
  rows1 = _edge_call(g1, eat, Wk1a, Wk1r, Wv1a, Wv1r)
  part1 = _scatter_call(dstc2d, rows1, zeros)

  tq2, tk2, tv2 = _combine1_call(part1, pos, Wq2, Wk2t, Wv2t)
  g2 = _gather_call(src2d, dst2d, tq2, tk2, tv2)
  rows2 = _edge_call(g2, eat, Wk2a, Wk2r, Wv2a, Wv2r)
  part2 = _scatter_call(dstc2d, rows2, zeros)
  return _combine2_call(part2)


# speedup vs baseline: 58.9530x; 1.2905x over previous
import functools

import jax
import jax.numpy as jnp
import numpy as np
from jax import lax
from jax.experimental import pallas as pl
from jax.experimental.pallas import tpu as pltpu
from jax.experimental.pallas import tpu_sc as plsc

N = 50000
E = 800000
D_IN = 15
D_EDGE = 5
NUM_RBF = 8
HID = 32
HEADS = 4
HEAD_DIM = HID // HEADS
CUTOFF = 8.0

NUM_CORES = 2
NUM_SUBCORES = 16
NW = NUM_CORES * NUM_SUBCORES
EPT = E // NW
MAC = 500
NMAC = EPT // MAC
SUB = 125
NSUB = MAC // SUB
AUG = 48
GW = 128
ACC_W = 48

HN = N // 2
NTRASH = 128
ER = E // 2
EPR = ER // NUM_SUBCORES
NMAC2 = EPR // MAC
ZR2 = (HN + NTRASH) // 8
OR2 = HN // 8

_GAMMA = CUTOFF / NUM_RBF
_SCALE = 1.0 / np.sqrt(float(HEAD_DIM))


@functools.cache
def _mesh():
  return plsc.VectorSubcoreMesh(
      core_axis_name="c", subcore_axis_name="s",
      num_cores=NUM_CORES, num_subcores=NUM_SUBCORES,
  )


def _centers_row():
  i = lax.broadcasted_iota(jnp.int32, (1, NUM_RBF), 1)
  return i.astype(jnp.float32) * (CUTOFF / (NUM_RBF - 1))


def _sel():
  r = lax.broadcasted_iota(jnp.int32, (HID, HEADS), 0) // HEAD_DIM
  c = lax.broadcasted_iota(jnp.int32, (HID, HEADS), 1)
  return (r == c).astype(jnp.float32)


def _selt():
  r = lax.broadcasted_iota(jnp.int32, (HEADS, HID), 0)
  c = lax.broadcasted_iota(jnp.int32, (HEADS, HID), 1) // HEAD_DIM
  return (r == c).astype(jnp.float32)


def _gather_call(src2d, dst2d, tq, tk, tv):
  widths = (AUG, AUG, HID)
  offs = (0, AUG, 2 * AUG)
  use_dst = (True, False, False)

  @functools.partial(
      pl.kernel,
      out_type=jax.ShapeDtypeStruct((E, GW), jnp.float32),
      mesh=_mesh(),
      compiler_params=pltpu.CompilerParams(use_tc_tiling_on_sc=False),
      scratch_types=[
          pltpu.VMEM((NSUB, SUB), jnp.int32),
          pltpu.VMEM((NSUB, SUB), jnp.int32),
          pltpu.VMEM((MAC, AUG), jnp.float32),
          pltpu.VMEM((MAC, AUG), jnp.float32),
          pltpu.VMEM((MAC, HID), jnp.float32),
          pltpu.SemaphoreType.DMA,
      ],
  )
  def gather_kernel(src2d_h, dst2d_h, tq_h, tk_h, tv_h, g_out,
                    idx_src, idx_dst, bq, bk, bv, sem):
    c = lax.axis_index("c")
    s = lax.axis_index("s")
    wid = s * NUM_CORES + c
    tables = (tq_h, tk_h, tv_h)
    bufs = (bq, bk, bv)

    def body(m, carry):
      base = wid * EPT + m * MAC
      r0 = wid * (EPT // SUB) + m * NSUB
      pltpu.sync_copy(src2d_h.at[pl.ds(r0, NSUB)], idx_src)
      pltpu.sync_copy(dst2d_h.at[pl.ds(r0, NSUB)], idx_dst)
      copies = []
      for j in range(NSUB):
        for tab, buf, dflag in zip(tables, bufs, use_dst):
          idx = (idx_dst if dflag else idx_src).at[j]
          copies.append(pltpu.async_copy(
              tab.at[idx], buf.at[pl.ds(j * SUB, SUB)], sem))
      for cp in copies:
        cp.wait()
      for buf, off, w in zip(bufs, offs, widths):
        pltpu.sync_copy(buf, g_out.at[pl.ds(base, MAC), pl.ds(off, w)])
      return carry

    lax.fori_loop(0, NMAC, body, 0)

  return gather_kernel(src2d, dst2d, tq, tk, tv)


def _scatter_call(dstc2d, rows, zeros):
  @functools.partial(
      pl.kernel,
      out_type=jax.ShapeDtypeStruct((N, ACC_W), jnp.float32),
      mesh=_mesh(),
      compiler_params=pltpu.CompilerParams(use_tc_tiling_on_sc=False),
      scratch_types=[
          pltpu.VMEM((NSUB, SUB), jnp.int32),
          pltpu.VMEM((NSUB, SUB), jnp.int32),
          pltpu.VMEM((MAC, ACC_W), jnp.float32),
          pltpu.VMEM((MAC, ACC_W), jnp.float32),
          pltpu.VMEM_SHARED((HN + NTRASH, ACC_W), jnp.float32),
          pltpu.SemaphoreType.DMA,
      ],
  )
  def scatter_kernel(dstc2d_h, rows_h, zeros_h, out, idx_a, idx_b,
                     rows_a, rows_b, acc, sem):
    c = lax.axis_index("c")
    s = lax.axis_index("s")

    @pl.when(s < 8)
    def _zero():
      pltpu.sync_copy(zeros_h, acc.at[pl.ds(s * ZR2, ZR2)])

    plsc.subcore_barrier()

    def body(m, carry):
      base = s * EPR + m * MAC
      r0 = s * (EPR // SUB) + m * NSUB
      pltpu.sync_copy(dstc2d_h.at[c, pl.ds(r0, NSUB)], idx_a)
      pltpu.sync_copy(dstc2d_h.at[c, pl.ds(r0 + ER // SUB, NSUB)], idx_b)
      pltpu.sync_copy(rows_h.at[pl.ds(base, MAC), pl.ds(0, ACC_W)], rows_a)
      pltpu.sync_copy(rows_h.at[pl.ds(base, MAC), pl.ds(ACC_W, ACC_W)],
                      rows_b)
      copies = []
      for j in range(NSUB):
        copies.append(pltpu.async_copy(
            rows_a.at[pl.ds(j * SUB, SUB)], acc.at[idx_a.at[j]], sem,
            add=True))
        copies.append(pltpu.async_copy(
            rows_b.at[pl.ds(j * SUB, SUB)], acc.at[idx_b.at[j]], sem,
            add=True))
      for cp in copies:
        cp.wait()
      return carry

    lax.fori_loop(0, NMAC2, body, 0)
    plsc.subcore_barrier()

    @pl.when(s < 8)
    def _out():
      pltpu.sync_copy(acc.at[pl.ds(s * OR2, OR2)],
                      out.at[pl.ds(c * HN + s * OR2, OR2)])

  return scatter_kernel(dstc2d, rows, zeros)


def _dstidx_call(dst):
  rows = E // 128

  def body(dst_ref, out_ref):
    dv = dst_ref[...]
    in0 = dv < HN
    trash = HN + (lax.broadcasted_iota(jnp.int32, dv.shape, 1)
                  % jnp.int32(NTRASH))
    out_ref[0] = jnp.where(in0, dv, trash)
    out_ref[1] = jnp.where(in0, trash, dv - HN)

  return pl.pallas_call(
      body,
      grid=(1,),
      in_specs=[pl.BlockSpec((rows, 128), lambda i: (0, 0))],
      out_specs=pl.BlockSpec((NUM_CORES, rows, 128), lambda i: (0, 0, 0)),
      out_shape=jax.ShapeDtypeStruct((NUM_CORES, rows, 128), jnp.int32),
  )(dst.reshape(rows, 128))


def _prep_call(x, pos, Wq, Wkt, Wvt):
  d_in = x.shape[1]
  bn = 5000

  def body(x_ref, pos_ref, wq, wk, wv, q_ref, k_ref, v_ref):
    xb = x_ref[...]
    pb = pos_ref[...]
    pad = jnp.zeros((bn, AUG - HID - 3), jnp.float32)
    q = jnp.dot(xb, wq[...], preferred_element_type=jnp.float32)
    k = jnp.dot(xb, wk[...], preferred_element_type=jnp.float32)
    q_ref[...] = jnp.concatenate([q, pb, pad], axis=1)
    k_ref[...] = jnp.concatenate([k, pb, pad], axis=1)
    v_ref[...] = jnp.dot(xb, wv[...], preferred_element_type=jnp.float32)

  return pl.pallas_call(
      body,
      grid=(N // bn,),
      in_specs=[
          pl.BlockSpec((bn, d_in), lambda i: (i, 0)),
          pl.BlockSpec((bn, 3), lambda i: (i, 0)),
          pl.BlockSpec((d_in, HID), lambda i: (0, 0)),
          pl.BlockSpec((d_in, HID), lambda i: (0, 0)),
          pl.BlockSpec((d_in, HID), lambda i: (0, 0)),
      ],
      out_specs=[
          pl.BlockSpec((bn, AUG), lambda i: (i, 0)),
          pl.BlockSpec((bn, AUG), lambda i: (i, 0)),
          pl.BlockSpec((bn, HID), lambda i: (i, 0)),
      ],
      out_shape=[
          jax.ShapeDtypeStruct((N, AUG), jnp.float32),
          jax.ShapeDtypeStruct((N, AUG), jnp.float32),
          jax.ShapeDtypeStruct((N, HID), jnp.float32),
      ],
  )(x, pos, Wq, Wkt, Wvt)


def _iota2(shape, dim):
  return lax.broadcasted_iota(jnp.int32, shape, dim)


def _mm(a, b):
  return jnp.dot(a, b, preferred_element_type=jnp.float32)


def _edge_rows(gb, eatb, wkva, wkvr, off):
  r128 = _iota2((GW, 4), 0)
  c4 = _iota2((GW, 4), 1)
  dif = ((r128 == c4 + HID).astype(jnp.float32)
         - (r128 == c4 + AUG + HID).astype(jnp.float32))
  r = _iota2((GW, 2 * HID), 0)
  c = _iota2((GW, 2 * HID), 1)
  sel_kv = (r == jnp.where(c < HID, c + AUG, c + 2 * AUG - HID)
            ).astype(jnp.float32)
  ones41 = jnp.ones((4, 1), jnp.float32)

  diff4 = _mm(gb, dif)
  dist = jnp.sqrt(_mm(diff4 * diff4, ones41) + 1e-9)
  rbf = jnp.exp(-((dist - _centers_row()) ** 2) / (_GAMMA ** 2))
  ea_kv = lax.dot_general(eatb, wkva, (((0,), (0,)), ((), ())),
                          preferred_element_type=jnp.float32)
  kv = _mm(gb, sel_kv) + ea_kv + _mm(rbf, wkvr)
  t32 = gb[:, 0:HID] * kv[:, 0:HID]
  p = jnp.exp(_mm(t32, _sel()) * _SCALE)
  pv32 = _mm(p, _selt()) * kv[:, HID:2 * HID]
  rp = _iota2((4, GW), 0)
  cp = _iota2((4, GW), 1)
  place_p = (cp == rp + off).astype(jnp.float32)
  rv = _iota2((HID, GW), 0)
  cv = _iota2((HID, GW), 1)
  place_v = (cv == rv + off + 8).astype(jnp.float32)
  return _mm(p, place_p) + _mm(pv32, place_v)


def _edge_call(g, eat, Wkva, Wkvr):
  be = 3200
  nblk = ER // be

  def body(ga_ref, gb_ref, eat_a, eat_b, wkva, wkvr, rows_ref):
    rows_ref[...] = (
        _edge_rows(ga_ref[...], eat_a[...], wkva[...], wkvr[...], 0)
        + _edge_rows(gb_ref[...], eat_b[...], wkva[...], wkvr[...], ACC_W))

  return pl.pallas_call(
      body,
      grid=(nblk,),
      in_specs=[
          pl.BlockSpec((be, GW), lambda i: (i, 0)),
          pl.BlockSpec((be, GW), lambda i: (i + nblk, 0)),
          pl.BlockSpec((D_EDGE, be), lambda i: (0, i)),
          pl.BlockSpec((D_EDGE, be), lambda i: (0, i + nblk)),
          pl.BlockSpec((D_EDGE, 2 * HID), lambda i: (0, 0)),
          pl.BlockSpec((NUM_RBF, 2 * HID), lambda i: (0, 0)),
      ],
      out_specs=pl.BlockSpec((be, GW), lambda i: (i, 0)),
      out_shape=jax.ShapeDtypeStruct((ER, GW), jnp.float32),
  )(g, g, eat, eat, Wkva, Wkvr)


def _combine1_call(part, pos, Wq, Wkt, Wvt):
  bn = 5000

  def body(part_ref, pos_ref, wq, wk, wv, q_ref, k_ref, v_ref):
    a = part_ref[...]
    den = jnp.dot(a[:, 0:4], _selt(), preferred_element_type=jnp.float32) + 1e-9
    h = jnp.maximum(a[:, 8:40] / den, 0.0)
    pb = pos_ref[...]
    pad = jnp.zeros((bn, AUG - HID - 3), jnp.float32)
    q = jnp.dot(h, wq[...], preferred_element_type=jnp.float32)
    k = jnp.dot(h, wk[...], preferred_element_type=jnp.float32)
    q_ref[...] = jnp.concatenate([q, pb, pad], axis=1)
    k_ref[...] = jnp.concatenate([k, pb, pad], axis=1)
    v_ref[...] = jnp.dot(h, wv[...], preferred_element_type=jnp.float32)

  return pl.pallas_call(
      body,
      grid=(N // bn,),
      in_specs=[
          pl.BlockSpec((bn, ACC_W), lambda i: (i, 0)),
          pl.BlockSpec((bn, 3), lambda i: (i, 0)),
          pl.BlockSpec((HID, HID), lambda i: (0, 0)),
          pl.BlockSpec((HID, HID), lambda i: (0, 0)),
          pl.BlockSpec((HID, HID), lambda i: (0, 0)),
      ],
      out_specs=[
          pl.BlockSpec((bn, AUG), lambda i: (i, 0)),
          pl.BlockSpec((bn, AUG), lambda i: (i, 0)),
          pl.BlockSpec((bn, HID), lambda i: (i, 0)),
      ],
      out_shape=[
          jax.ShapeDtypeStruct((N, AUG), jnp.float32),
          jax.ShapeDtypeStruct((N, AUG), jnp.float32),
          jax.ShapeDtypeStruct((N, HID), jnp.float32),
      ],
  )(part, pos, Wq, Wkt, Wvt)


def _combine2_call(part):
  bn = 5000

  def body(part_ref, h_ref):
    a = part_ref[...]
    den = jnp.dot(a[:, 0:4], _selt(), preferred_element_type=jnp.float32) + 1e-9
    h_ref[...] = a[:, 8:40] / den

  return pl.pallas_call(
      body,
      grid=(N // bn,),
      in_specs=[pl.BlockSpec((bn, ACC_W), lambda i: (i, 0))],
      out_specs=pl.BlockSpec((bn, HID), lambda i: (i, 0)),
      out_shape=jax.ShapeDtypeStruct((N, HID), jnp.float32),
  )(part)


def kernel(x, edge_index, edge_attr, pos, Wq1, Wk1, Wv1, Wq2, Wk2, Wv2):
  src2d = edge_index[0].reshape(E // SUB, SUB)
  dst2d = edge_index[1].reshape(E // SUB, SUB)
  eat = edge_attr.T

  Wk1t, Wk1a, Wk1r = Wk1[:D_IN], Wk1[D_IN:D_IN + D_EDGE], Wk1[D_IN + D_EDGE:]
  Wv1t, Wv1a, Wv1r = Wv1[:D_IN], Wv1[D_IN:D_IN + D_EDGE], Wv1[D_IN + D_EDGE:]
  Wk2t, Wk2a, Wk2r = Wk2[:HID], Wk2[HID:HID + D_EDGE], Wk2[HID + D_EDGE:]
  Wv2t, Wv2a, Wv2r = Wv2[:HID], Wv2[HID:HID + D_EDGE], Wv2[HID + D_EDGE:]

  zeros = jnp.zeros((ZR2, ACC_W), jnp.float32)
  dstc2d = _dstidx_call(edge_index[1]).reshape(NUM_CORES, E // SUB, SUB)

  Wkv1a = jnp.concatenate([Wk1a, Wv1a], axis=1)
  Wkv1r = jnp.concatenate([Wk1r, Wv1r], axis=1)
  Wkv2a = jnp.concatenate([Wk2a, Wv2a], axis=1)
  Wkv2r = jnp.concatenate([Wk2r, Wv2r], axis=1)

  tq1, tk1, tv1 = _prep_call(x, pos, Wq1, Wk1t, Wv1t)
  g1 = _gather_call(src2d, dst2d, tq1, tk1, tv1)
  rows1 = _edge_call(g1, eat, Wkv1a, Wkv1r)
  part1 = _scatter_call(dstc2d, rows1, zeros)

  tq2, tk2, tv2 = _combine1_call(part1, pos, Wq2, Wk2t, Wv2t)
  g2 = _gather_call(src2d, dst2d, tq2, tk2, tv2)
  rows2 = _edge_call(g2, eat, Wkv2a, Wkv2r)
  part2 = _scatter_call(dstc2d, rows2, zeros)
  return _combine2_call(part2)


# speedup vs baseline: 62.0025x; 1.0517x over previous
import functools

import jax
import jax.numpy as jnp
import numpy as np
from jax import lax
from jax.experimental import pallas as pl
from jax.experimental.pallas import tpu as pltpu
from jax.experimental.pallas import tpu_sc as plsc

N = 50000
E = 800000
D_IN = 15
D_EDGE = 5
NUM_RBF = 8
HID = 32
HEADS = 4
HEAD_DIM = HID // HEADS
CUTOFF = 8.0

NUM_CORES = 2
NUM_SUBCORES = 16
NW = NUM_CORES * NUM_SUBCORES
EPT = E // NW
GMAC = 250
NGMAC = EPT // GMAC
MAC = 500
SUB = 125
NSUB = MAC // SUB
GSUB = GMAC // SUB
AUG = 48
GW = 128
ACC_W = 48

HN = N // 2
NTRASH = 128
ER = E // 2
EPR = ER // NUM_SUBCORES
NMAC2 = EPR // MAC
ZR2 = (HN + NTRASH) // 8
OR2 = HN // 8

_GAMMA = CUTOFF / NUM_RBF
_SCALE = 1.0 / np.sqrt(float(HEAD_DIM))


@functools.cache
def _mesh():
  return plsc.VectorSubcoreMesh(
      core_axis_name="c", subcore_axis_name="s",
      num_cores=NUM_CORES, num_subcores=NUM_SUBCORES,
  )


def _centers_row():
  i = lax.broadcasted_iota(jnp.int32, (1, NUM_RBF), 1)
  return i.astype(jnp.float32) * (CUTOFF / (NUM_RBF - 1))


def _sel():
  r = lax.broadcasted_iota(jnp.int32, (HID, HEADS), 0) // HEAD_DIM
  c = lax.broadcasted_iota(jnp.int32, (HID, HEADS), 1)
  return (r == c).astype(jnp.float32)


def _selt():
  r = lax.broadcasted_iota(jnp.int32, (HEADS, HID), 0)
  c = lax.broadcasted_iota(jnp.int32, (HEADS, HID), 1) // HEAD_DIM
  return (r == c).astype(jnp.float32)


def _gather_call(src2d, dst2d, tq, tk, tv):
  widths = (AUG, AUG, HID)
  offs = (0, AUG, 2 * AUG)
  use_dst = (True, False, False)

  @functools.partial(
      pl.kernel,
      out_type=jax.ShapeDtypeStruct((E, GW), jnp.float32),
      mesh=_mesh(),
      compiler_params=pltpu.CompilerParams(use_tc_tiling_on_sc=False),
      scratch_types=[
          pltpu.VMEM((GSUB, SUB), jnp.int32),
          pltpu.VMEM((GSUB, SUB), jnp.int32),
          pltpu.VMEM((GSUB, SUB), jnp.int32),
          pltpu.VMEM((GSUB, SUB), jnp.int32),
          pltpu.VMEM((GMAC, AUG), jnp.float32),
          pltpu.VMEM((GMAC, AUG), jnp.float32),
          pltpu.VMEM((GMAC, HID), jnp.float32),
          pltpu.VMEM((GMAC, AUG), jnp.float32),
          pltpu.VMEM((GMAC, AUG), jnp.float32),
          pltpu.VMEM((GMAC, HID), jnp.float32),
          pltpu.SemaphoreType.DMA,
          pltpu.SemaphoreType.DMA,
          pltpu.SemaphoreType.DMA,
          pltpu.SemaphoreType.DMA,
      ],
  )
  def gather_kernel(src2d_h, dst2d_h, tq_h, tk_h, tv_h, g_out,
                    is0, id0, is1, id1, bq0, bk0, bv0, bq1, bk1, bv1,
                    gsem0, gsem1, wsem0, wsem1):
    c = lax.axis_index("c")
    s = lax.axis_index("s")
    wid = s * NUM_CORES + c
    tables = (tq_h, tk_h, tv_h)
    slot_bufs = ((bq0, bk0, bv0), (bq1, bk1, bv1))
    slot_idx = ((is0, id0), (is1, id1))
    gsems = (gsem0, gsem1)
    wsems = (wsem0, wsem1)

    def fire(m, slot):
      r0 = wid * (EPT // SUB) + m * GSUB
      isrc, idst = slot_idx[slot]
      pltpu.sync_copy(src2d_h.at[pl.ds(r0, GSUB)], isrc)
      pltpu.sync_copy(dst2d_h.at[pl.ds(r0, GSUB)], idst)
      copies = []
      for j in range(GSUB):
        for tab, buf, dflag in zip(tables, slot_bufs[slot], use_dst):
          idx = (idst if dflag else isrc).at[j]
          copies.append(pltpu.async_copy(
              tab.at[idx], buf.at[pl.ds(j * SUB, SUB)], gsems[slot]))
      return copies

    def fire_writes(m, slot):
      base = wid * EPT + m * GMAC
      for buf, off, w in zip(slot_bufs[slot], offs, widths):
        pltpu.async_copy(buf, g_out.at[pl.ds(base, GMAC), pl.ds(off, w)],
                         wsems[slot])

    def drain_writes(m, slot):
      base = wid * EPT + m * GMAC
      for buf, off, w in zip(slot_bufs[slot], offs, widths):
        pltpu.make_async_copy(
            buf, g_out.at[pl.ds(base, GMAC), pl.ds(off, w)],
            wsems[slot]).wait()

    def body(i, carry):
      m0 = 2 * i
      m1 = 2 * i + 1

      @pl.when(i > 0)
      def _():
        drain_writes(m0 - 2, 0)

      g0 = fire(m0, 0)

      @pl.when(i > 0)
      def _():
        drain_writes(m1 - 2, 1)

      g1 = fire(m1, 1)
      for cp in g0:
        cp.wait()
      fire_writes(m0, 0)
      for cp in g1:
        cp.wait()
      fire_writes(m1, 1)
      return carry

    lax.fori_loop(0, NGMAC // 2, body, 0)
    drain_writes(NGMAC - 2, 0)
    drain_writes(NGMAC - 1, 1)

  return gather_kernel(src2d, dst2d, tq, tk, tv)


def _scatter_call(dstc2d, rows, zeros):
  @functools.partial(
      pl.kernel,
      out_type=jax.ShapeDtypeStruct((N, ACC_W), jnp.float32),
      mesh=_mesh(),
      compiler_params=pltpu.CompilerParams(use_tc_tiling_on_sc=False),
      scratch_types=[
          pltpu.VMEM((NSUB, SUB), jnp.int32),
          pltpu.VMEM((NSUB, SUB), jnp.int32),
          pltpu.VMEM((MAC, ACC_W), jnp.float32),
          pltpu.VMEM((MAC, ACC_W), jnp.float32),
          pltpu.VMEM_SHARED((HN + NTRASH, ACC_W), jnp.float32),
          pltpu.SemaphoreType.DMA,
      ],
  )
  def scatter_kernel(dstc2d_h, rows_h, zeros_h, out, idx_a, idx_b,
                     rows_a, rows_b, acc, sem):
    c = lax.axis_index("c")
    s = lax.axis_index("s")

    @pl.when(s < 8)
    def _zero():
      pltpu.sync_copy(zeros_h, acc.at[pl.ds(s * ZR2, ZR2)])

    plsc.subcore_barrier()

    def body(m, carry):
      base = s * EPR + m * MAC
      r0 = s * (EPR // SUB) + m * NSUB
      pltpu.sync_copy(dstc2d_h.at[c, pl.ds(r0, NSUB)], idx_a)
      pltpu.sync_copy(dstc2d_h.at[c, pl.ds(r0 + ER // SUB, NSUB)], idx_b)
      pltpu.sync_copy(rows_h.at[pl.ds(base, MAC), pl.ds(0, ACC_W)], rows_a)
      pltpu.sync_copy(rows_h.at[pl.ds(base, MAC), pl.ds(ACC_W, ACC_W)],
                      rows_b)
      copies = []
      for j in range(NSUB):
        copies.append(pltpu.async_copy(
            rows_a.at[pl.ds(j * SUB, SUB)], acc.at[idx_a.at[j]], sem,
            add=True))
        copies.append(pltpu.async_copy(
            rows_b.at[pl.ds(j * SUB, SUB)], acc.at[idx_b.at[j]], sem,
            add=True))
      for cp in copies:
        cp.wait()
      return carry

    lax.fori_loop(0, NMAC2, body, 0)
    plsc.subcore_barrier()

    @pl.when(s < 8)
    def _out():
      pltpu.sync_copy(acc.at[pl.ds(s * OR2, OR2)],
                      out.at[pl.ds(c * HN + s * OR2, OR2)])

  return scatter_kernel(dstc2d, rows, zeros)


def _dstidx_call(dst):
  rows = E // 128

  def body(dst_ref, out_ref):
    dv = dst_ref[...]
    in0 = dv < HN
    trash = HN + (lax.broadcasted_iota(jnp.int32, dv.shape, 1)
                  % jnp.int32(NTRASH))
    out_ref[0] = jnp.where(in0, dv, trash)
    out_ref[1] = jnp.where(in0, trash, dv - HN)

  return pl.pallas_call(
      body,
      grid=(1,),
      in_specs=[pl.BlockSpec((rows, 128), lambda i: (0, 0))],
      out_specs=pl.BlockSpec((NUM_CORES, rows, 128), lambda i: (0, 0, 0)),
      out_shape=jax.ShapeDtypeStruct((NUM_CORES, rows, 128), jnp.int32),
  )(dst.reshape(rows, 128))


def _prep_call(x, pos, Wq, Wkt, Wvt):
  d_in = x.shape[1]
  bn = 5000

  def body(x_ref, pos_ref, wq, wk, wv, q_ref, k_ref, v_ref):
    xb = x_ref[...]
    pb = pos_ref[...]
    pad = jnp.zeros((bn, AUG - HID - 3), jnp.float32)
    q = jnp.dot(xb, wq[...], preferred_element_type=jnp.float32)
    k = jnp.dot(xb, wk[...], preferred_element_type=jnp.float32)
    q_ref[...] = jnp.concatenate([q, pb, pad], axis=1)
    k_ref[...] = jnp.concatenate([k, pb, pad], axis=1)
    v_ref[...] = jnp.dot(xb, wv[...], preferred_element_type=jnp.float32)

  return pl.pallas_call(
      body,
      grid=(N // bn,),
      in_specs=[
          pl.BlockSpec((bn, d_in), lambda i: (i, 0)),
          pl.BlockSpec((bn, 3), lambda i: (i, 0)),
          pl.BlockSpec((d_in, HID), lambda i: (0, 0)),
          pl.BlockSpec((d_in, HID), lambda i: (0, 0)),
          pl.BlockSpec((d_in, HID), lambda i: (0, 0)),
      ],
      out_specs=[
          pl.BlockSpec((bn, AUG), lambda i: (i, 0)),
          pl.BlockSpec((bn, AUG), lambda i: (i, 0)),
          pl.BlockSpec((bn, HID), lambda i: (i, 0)),
      ],
      out_shape=[
          jax.ShapeDtypeStruct((N, AUG), jnp.float32),
          jax.ShapeDtypeStruct((N, AUG), jnp.float32),
          jax.ShapeDtypeStruct((N, HID), jnp.float32),
      ],
  )(x, pos, Wq, Wkt, Wvt)


def _iota2(shape, dim):
  return lax.broadcasted_iota(jnp.int32, shape, dim)


def _mm(a, b):
  return jnp.dot(a, b, preferred_element_type=jnp.float32)


def _edge_rows(gb, eatb, wkva, wkvr, off):
  r128 = _iota2((GW, 4), 0)
  c4 = _iota2((GW, 4), 1)
  dif = ((r128 == c4 + HID).astype(jnp.float32)
         - (r128 == c4 + AUG + HID).astype(jnp.float32))
  r = _iota2((GW, 2 * HID), 0)
  c = _iota2((GW, 2 * HID), 1)
  sel_kv = (r == jnp.where(c < HID, c + AUG, c + 2 * AUG - HID)
            ).astype(jnp.float32)
  ones41 = jnp.ones((4, 1), jnp.float32)

  diff4 = _mm(gb, dif)
  dist = jnp.sqrt(_mm(diff4 * diff4, ones41) + 1e-9)
  rbf = jnp.exp(-((dist - _centers_row()) ** 2) / (_GAMMA ** 2))
  ea_kv = lax.dot_general(eatb, wkva, (((0,), (0,)), ((), ())),
                          preferred_element_type=jnp.float32)
  kv = _mm(gb, sel_kv) + ea_kv + _mm(rbf, wkvr)
  t32 = gb[:, 0:HID] * kv[:, 0:HID]
  p = jnp.exp(_mm(t32, _sel()) * _SCALE)
  pv32 = _mm(p, _selt()) * kv[:, HID:2 * HID]
  rp = _iota2((4, GW), 0)
  cp = _iota2((4, GW), 1)
  place_p = (cp == rp + off).astype(jnp.float32)
  rv = _iota2((HID, GW), 0)
  cv = _iota2((HID, GW), 1)
  place_v = (cv == rv + off + 8).astype(jnp.float32)
  return _mm(p, place_p) + _mm(pv32, place_v)


def _edge_call(g, eat, Wkva, Wkvr):
  be = 3200
  nblk = ER // be

  def body(ga_ref, gb_ref, eat_a, eat_b, wkva, wkvr, rows_ref):
    rows_ref[...] = (
        _edge_rows(ga_ref[...], eat_a[...], wkva[...], wkvr[...], 0)
        + _edge_rows(gb_ref[...], eat_b[...], wkva[...], wkvr[...], ACC_W))

  return pl.pallas_call(
      body,
      grid=(nblk,),
      in_specs=[
          pl.BlockSpec((be, GW), lambda i: (i, 0)),
          pl.BlockSpec((be, GW), lambda i: (i + nblk, 0)),
          pl.BlockSpec((D_EDGE, be), lambda i: (0, i)),
          pl.BlockSpec((D_EDGE, be), lambda i: (0, i + nblk)),
          pl.BlockSpec((D_EDGE, 2 * HID), lambda i: (0, 0)),
          pl.BlockSpec((NUM_RBF, 2 * HID), lambda i: (0, 0)),
      ],
      out_specs=pl.BlockSpec((be, GW), lambda i: (i, 0)),
      out_shape=jax.ShapeDtypeStruct((ER, GW), jnp.float32),
  )(g, g, eat, eat, Wkva, Wkvr)


def _combine1_call(part, pos, Wq, Wkt, Wvt):
  bn = 5000

  def body(part_ref, pos_ref, wq, wk, wv, q_ref, k_ref, v_ref):
    a = part_ref[...]
    den = jnp.dot(a[:, 0:4], _selt(), preferred_element_type=jnp.float32) + 1e-9
    h = jnp.maximum(a[:, 8:40] / den, 0.0)
    pb = pos_ref[...]
    pad = jnp.zeros((bn, AUG - HID - 3), jnp.float32)
    q = jnp.dot(h, wq[...], preferred_element_type=jnp.float32)
    k = jnp.dot(h, wk[...], preferred_element_type=jnp.float32)
    q_ref[...] = jnp.concatenate([q, pb, pad], axis=1)
    k_ref[...] = jnp.concatenate([k, pb, pad], axis=1)
    v_ref[...] = jnp.dot(h, wv[...], preferred_element_type=jnp.float32)

  return pl.pallas_call(
      body,
      grid=(N // bn,),
      in_specs=[
          pl.BlockSpec((bn, ACC_W), lambda i: (i, 0)),
          pl.BlockSpec((bn, 3), lambda i: (i, 0)),
          pl.BlockSpec((HID, HID), lambda i: (0, 0)),
          pl.BlockSpec((HID, HID), lambda i: (0, 0)),
          pl.BlockSpec((HID, HID), lambda i: (0, 0)),
      ],
      out_specs=[
          pl.BlockSpec((bn, AUG), lambda i: (i, 0)),
          pl.BlockSpec((bn, AUG), lambda i: (i, 0)),
          pl.BlockSpec((bn, HID), lambda i: (i, 0)),
      ],
      out_shape=[
          jax.ShapeDtypeStruct((N, AUG), jnp.float32),
          jax.ShapeDtypeStruct((N, AUG), jnp.float32),
          jax.ShapeDtypeStruct((N, HID), jnp.float32),
      ],
  )(part, pos, Wq, Wkt, Wvt)


def _combine2_call(part):
  bn = 5000

  def body(part_ref, h_ref):
    a = part_ref[...]
    den = jnp.dot(a[:, 0:4], _selt(), preferred_element_type=jnp.float32) + 1e-9
    h_ref[...] = a[:, 8:40] / den

  return pl.pallas_call(
      body,
      grid=(N // bn,),
      in_specs=[pl.BlockSpec((bn, ACC_W), lambda i: (i, 0))],
      out_specs=pl.BlockSpec((bn, HID), lambda i: (i, 0)),
      out_shape=jax.ShapeDtypeStruct((N, HID), jnp.float32),
  )(part)


def kernel(x, edge_index, edge_attr, pos, Wq1, Wk1, Wv1, Wq2, Wk2, Wv2):
  src2d = edge_index[0].reshape(E // SUB, SUB)
  dst2d = edge_index[1].reshape(E // SUB, SUB)
  eat = edge_attr.T

  Wk1t, Wk1a, Wk1r = Wk1[:D_IN], Wk1[D_IN:D_IN + D_EDGE], Wk1[D_IN + D_EDGE:]
  Wv1t, Wv1a, Wv1r = Wv1[:D_IN], Wv1[D_IN:D_IN + D_EDGE], Wv1[D_IN + D_EDGE:]
  Wk2t, Wk2a, Wk2r = Wk2[:HID], Wk2[HID:HID + D_EDGE], Wk2[HID + D_EDGE:]
  Wv2t, Wv2a, Wv2r = Wv2[:HID], Wv2[HID:HID + D_EDGE], Wv2[HID + D_EDGE:]

  zeros = jnp.zeros((ZR2, ACC_W), jnp.float32)
  dstc2d = _dstidx_call(edge_index[1]).reshape(NUM_CORES, E // SUB, SUB)

  Wkv1a = jnp.concatenate([Wk1a, Wv1a], axis=1)
  Wkv1r = jnp.concatenate([Wk1r, Wv1r], axis=1)
  Wkv2a = jnp.concatenate([Wk2a, Wv2a], axis=1)
  Wkv2r = jnp.concatenate([Wk2r, Wv2r], axis=1)

  tq1, tk1, tv1 = _prep_call(x, pos, Wq1, Wk1t, Wv1t)
  g1 = _gather_call(src2d, dst2d, tq1, tk1, tv1)
  rows1 = _edge_call(g1, eat, Wkv1a, Wkv1r)
  part1 = _scatter_call(dstc2d, rows1, zeros)

  tq2, tk2, tv2 = _combine1_call(part1, pos, Wq2, Wk2t, Wv2t)
  g2 = _gather_call(src2d, dst2d, tq2, tk2, tv2)
  rows2 = _edge_call(g2, eat, Wkv2a, Wkv2r)
  part2 = _scatter_call(dstc2d, rows2, zeros)
  return _combine2_call(part2)


# speedup vs baseline: 65.7876x; 1.0610x over previous
import functools

import jax
import jax.numpy as jnp
import numpy as np
from jax import lax
from jax.experimental import pallas as pl
from jax.experimental.pallas import tpu as pltpu
from jax.experimental.pallas import tpu_sc as plsc

N = 50000
E = 800000
D_IN = 15
D_EDGE = 5
NUM_RBF = 8
HID = 32
HEADS = 4
HEAD_DIM = HID // HEADS
CUTOFF = 8.0

NUM_CORES = 2
NUM_SUBCORES = 16
NW = NUM_CORES * NUM_SUBCORES
EPT = E // NW
GMAC = 250
NGMAC = EPT // GMAC
MAC = 250
SUB = 125
NSUB = MAC // SUB
GSUB = GMAC // SUB
AUG = 48
GW = 128
ACC_W = 48

HN = N // 2
NTRASH = 128
ER = E // 2
EPR = ER // NUM_SUBCORES
NMAC2 = EPR // MAC
ZR2 = (HN + NTRASH) // 8
OR2 = HN // 8

_GAMMA = CUTOFF / NUM_RBF
_SCALE = 1.0 / np.sqrt(float(HEAD_DIM))


@functools.cache
def _mesh():
  return plsc.VectorSubcoreMesh(
      core_axis_name="c", subcore_axis_name="s",
      num_cores=NUM_CORES, num_subcores=NUM_SUBCORES,
  )


def _centers_row():
  i = lax.broadcasted_iota(jnp.int32, (1, NUM_RBF), 1)
  return i.astype(jnp.float32) * (CUTOFF / (NUM_RBF - 1))


def _sel():
  r = lax.broadcasted_iota(jnp.int32, (HID, HEADS), 0) // HEAD_DIM
  c = lax.broadcasted_iota(jnp.int32, (HID, HEADS), 1)
  return (r == c).astype(jnp.float32)


def _selt():
  r = lax.broadcasted_iota(jnp.int32, (HEADS, HID), 0)
  c = lax.broadcasted_iota(jnp.int32, (HEADS, HID), 1) // HEAD_DIM
  return (r == c).astype(jnp.float32)


def _gather_call(src2d, dst2d, tq, tk, tv):
  widths = (AUG, AUG, HID)
  offs = (0, AUG, 2 * AUG)
  use_dst = (True, False, False)

  @functools.partial(
      pl.kernel,
      out_type=jax.ShapeDtypeStruct((E, GW), jnp.float32),
      mesh=_mesh(),
      compiler_params=pltpu.CompilerParams(use_tc_tiling_on_sc=False),
      scratch_types=[
          pltpu.VMEM((GSUB, SUB), jnp.int32),
          pltpu.VMEM((GSUB, SUB), jnp.int32),
          pltpu.VMEM((GSUB, SUB), jnp.int32),
          pltpu.VMEM((GSUB, SUB), jnp.int32),
          pltpu.VMEM((GMAC, AUG), jnp.float32),
          pltpu.VMEM((GMAC, AUG), jnp.float32),
          pltpu.VMEM((GMAC, HID), jnp.float32),
          pltpu.VMEM((GMAC, AUG), jnp.float32),
          pltpu.VMEM((GMAC, AUG), jnp.float32),
          pltpu.VMEM((GMAC, HID), jnp.float32),
          pltpu.SemaphoreType.DMA,
          pltpu.SemaphoreType.DMA,
          pltpu.SemaphoreType.DMA,
          pltpu.SemaphoreType.DMA,
      ],
  )
  def gather_kernel(src2d_h, dst2d_h, tq_h, tk_h, tv_h, g_out,
                    is0, id0, is1, id1, bq0, bk0, bv0, bq1, bk1, bv1,
                    gsem0, gsem1, wsem0, wsem1):
    c = lax.axis_index("c")
    s = lax.axis_index("s")
    wid = s * NUM_CORES + c
    tables = (tq_h, tk_h, tv_h)
    slot_bufs = ((bq0, bk0, bv0), (bq1, bk1, bv1))
    slot_idx = ((is0, id0), (is1, id1))
    gsems = (gsem0, gsem1)
    wsems = (wsem0, wsem1)

    def fire(m, slot):
      r0 = wid * (EPT // SUB) + m * GSUB
      isrc, idst = slot_idx[slot]
      pltpu.sync_copy(src2d_h.at[pl.ds(r0, GSUB)], isrc)
      pltpu.sync_copy(dst2d_h.at[pl.ds(r0, GSUB)], idst)
      copies = []
      for j in range(GSUB):
        for tab, buf, dflag in zip(tables, slot_bufs[slot], use_dst):
          idx = (idst if dflag else isrc).at[j]
          copies.append(pltpu.async_copy(
              tab.at[idx], buf.at[pl.ds(j * SUB, SUB)], gsems[slot]))
      return copies

    def fire_writes(m, slot):
      base = wid * EPT + m * GMAC
      for buf, off, w in zip(slot_bufs[slot], offs, widths):
        pltpu.async_copy(buf, g_out.at[pl.ds(base, GMAC), pl.ds(off, w)],
                         wsems[slot])

    def drain_writes(m, slot):
      base = wid * EPT + m * GMAC
      for buf, off, w in zip(slot_bufs[slot], offs, widths):
        pltpu.make_async_copy(
            buf, g_out.at[pl.ds(base, GMAC), pl.ds(off, w)],
            wsems[slot]).wait()

    def body(i, carry):
      m0 = 2 * i
      m1 = 2 * i + 1

      @pl.when(i > 0)
      def _():
        drain_writes(m0 - 2, 0)

      g0 = fire(m0, 0)

      @pl.when(i > 0)
      def _():
        drain_writes(m1 - 2, 1)

      g1 = fire(m1, 1)
      for cp in g0:
        cp.wait()
      fire_writes(m0, 0)
      for cp in g1:
        cp.wait()
      fire_writes(m1, 1)
      return carry

    lax.fori_loop(0, NGMAC // 2, body, 0)
    drain_writes(NGMAC - 2, 0)
    drain_writes(NGMAC - 1, 1)

  return gather_kernel(src2d, dst2d, tq, tk, tv)


def _scatter_call(dstc2d, rows, zeros):
  @functools.partial(
      pl.kernel,
      out_type=jax.ShapeDtypeStruct((N, ACC_W), jnp.float32),
      mesh=_mesh(),
      compiler_params=pltpu.CompilerParams(use_tc_tiling_on_sc=False),
      scratch_types=[
          pltpu.VMEM((NSUB, SUB), jnp.int32),
          pltpu.VMEM((NSUB, SUB), jnp.int32),
          pltpu.VMEM((NSUB, SUB), jnp.int32),
          pltpu.VMEM((NSUB, SUB), jnp.int32),
          pltpu.VMEM((MAC, ACC_W), jnp.float32),
          pltpu.VMEM((MAC, ACC_W), jnp.float32),
          pltpu.VMEM((MAC, ACC_W), jnp.float32),
          pltpu.VMEM((MAC, ACC_W), jnp.float32),
          pltpu.VMEM_SHARED((HN + NTRASH, ACC_W), jnp.float32),
          pltpu.SemaphoreType.DMA,
          pltpu.SemaphoreType.DMA,
          pltpu.SemaphoreType.DMA,
      ],
  )
  def scatter_kernel(dstc2d_h, rows_h, zeros_h, out, ia0, ib0, ia1, ib1,
                     ra0, rb0, ra1, rb1, acc, lsem0, lsem1, asem):
    c = lax.axis_index("c")
    s = lax.axis_index("s")
    slot_refs = ((ia0, ib0, ra0, rb0, lsem0), (ia1, ib1, ra1, rb1, lsem1))

    @pl.when(s < 8)
    def _zero():
      pltpu.sync_copy(zeros_h, acc.at[pl.ds(s * ZR2, ZR2)])

    plsc.subcore_barrier()

    def fire_loads(m, slot):
      ia, ib, ra, rb, lsem = slot_refs[slot]
      base = s * EPR + m * MAC
      r0 = s * (EPR // SUB) + m * NSUB
      return [
          pltpu.async_copy(dstc2d_h.at[c, pl.ds(r0, NSUB)], ia, lsem),
          pltpu.async_copy(dstc2d_h.at[c, pl.ds(r0 + ER // SUB, NSUB)], ib,
                           lsem),
          pltpu.async_copy(rows_h.at[pl.ds(base, MAC), pl.ds(0, ACC_W)], ra,
                           lsem),
          pltpu.async_copy(rows_h.at[pl.ds(base, MAC), pl.ds(ACC_W, ACC_W)],
                           rb, lsem),
      ]

    def fire_adds(slot):
      ia, ib, ra, rb, _ = slot_refs[slot]
      copies = []
      for j in range(NSUB):
        copies.append(pltpu.async_copy(
            ra.at[pl.ds(j * SUB, SUB)], acc.at[ia.at[j]], asem, add=True))
        copies.append(pltpu.async_copy(
            rb.at[pl.ds(j * SUB, SUB)], acc.at[ib.at[j]], asem, add=True))
      return copies

    def body(i, carry):
      l0 = fire_loads(2 * i, 0)
      l1 = fire_loads(2 * i + 1, 1)
      for cp in l0:
        cp.wait()
      a0 = fire_adds(0)
      for cp in l1:
        cp.wait()
      a1 = fire_adds(1)
      for cp in a0:
        cp.wait()
      for cp in a1:
        cp.wait()
      return carry

    lax.fori_loop(0, NMAC2 // 2, body, 0)
    plsc.subcore_barrier()

    @pl.when(s < 8)
    def _out():
      pltpu.sync_copy(acc.at[pl.ds(s * OR2, OR2)],
                      out.at[pl.ds(c * HN + s * OR2, OR2)])

  return scatter_kernel(dstc2d, rows, zeros)


def _dstidx_call(dst):
  rows = E // 128

  def body(dst_ref, out_ref):
    dv = dst_ref[...]
    in0 = dv < HN
    trash = HN + (lax.broadcasted_iota(jnp.int32, dv.shape, 1)
                  % jnp.int32(NTRASH))
    out_ref[0] = jnp.where(in0, dv, trash)
    out_ref[1] = jnp.where(in0, trash, dv - HN)

  return pl.pallas_call(
      body,
      grid=(1,),
      in_specs=[pl.BlockSpec((rows, 128), lambda i: (0, 0))],
      out_specs=pl.BlockSpec((NUM_CORES, rows, 128), lambda i: (0, 0, 0)),
      out_shape=jax.ShapeDtypeStruct((NUM_CORES, rows, 128), jnp.int32),
  )(dst.reshape(rows, 128))


def _prep_call(x, pos, Wq, Wkt, Wvt):
  d_in = x.shape[1]
  bn = 5000

  def body(x_ref, pos_ref, wq, wk, wv, q_ref, k_ref, v_ref):
    xb = x_ref[...]
    pb = pos_ref[...]
    pad = jnp.zeros((bn, AUG - HID - 3), jnp.float32)
    q = jnp.dot(xb, wq[...], preferred_element_type=jnp.float32)
    k = jnp.dot(xb, wk[...], preferred_element_type=jnp.float32)
    q_ref[...] = jnp.concatenate([q, pb, pad], axis=1)
    k_ref[...] = jnp.concatenate([k, pb, pad], axis=1)
    v_ref[...] = jnp.dot(xb, wv[...], preferred_element_type=jnp.float32)

  return pl.pallas_call(
      body,
      grid=(N // bn,),
      in_specs=[
          pl.BlockSpec((bn, d_in), lambda i: (i, 0)),
          pl.BlockSpec((bn, 3), lambda i: (i, 0)),
          pl.BlockSpec((d_in, HID), lambda i: (0, 0)),
          pl.BlockSpec((d_in, HID), lambda i: (0, 0)),
          pl.BlockSpec((d_in, HID), lambda i: (0, 0)),
      ],
      out_specs=[
          pl.BlockSpec((bn, AUG), lambda i: (i, 0)),
          pl.BlockSpec((bn, AUG), lambda i: (i, 0)),
          pl.BlockSpec((bn, HID), lambda i: (i, 0)),
      ],
      out_shape=[
          jax.ShapeDtypeStruct((N, AUG), jnp.float32),
          jax.ShapeDtypeStruct((N, AUG), jnp.float32),
          jax.ShapeDtypeStruct((N, HID), jnp.float32),
      ],
  )(x, pos, Wq, Wkt, Wvt)


def _iota2(shape, dim):
  return lax.broadcasted_iota(jnp.int32, shape, dim)


def _mm(a, b):
  return jnp.dot(a, b, preferred_element_type=jnp.float32)


def _edge_rows(gb, eatb, wkva, wkvr, off):
  r128 = _iota2((GW, 4), 0)
  c4 = _iota2((GW, 4), 1)
  dif = ((r128 == c4 + HID).astype(jnp.float32)
         - (r128 == c4 + AUG + HID).astype(jnp.float32))
  r = _iota2((GW, 2 * HID), 0)
  c = _iota2((GW, 2 * HID), 1)
  sel_kv = (r == jnp.where(c < HID, c + AUG, c + 2 * AUG - HID)
            ).astype(jnp.float32)
  ones41 = jnp.ones((4, 1), jnp.float32)

  diff4 = _mm(gb, dif)
  dist = jnp.sqrt(_mm(diff4 * diff4, ones41) + 1e-9)
  rbf = jnp.exp(-((dist - _centers_row()) ** 2) / (_GAMMA ** 2))
  ea_kv = lax.dot_general(eatb, wkva, (((0,), (0,)), ((), ())),
                          preferred_element_type=jnp.float32)
  kv = _mm(gb, sel_kv) + ea_kv + _mm(rbf, wkvr)
  t32 = gb[:, 0:HID] * kv[:, 0:HID]
  p = jnp.exp(_mm(t32, _sel()) * _SCALE)
  pv32 = _mm(p, _selt()) * kv[:, HID:2 * HID]
  rp = _iota2((4, GW), 0)
  cp = _iota2((4, GW), 1)
  place_p = (cp == rp + off).astype(jnp.float32)
  rv = _iota2((HID, GW), 0)
  cv = _iota2((HID, GW), 1)
  place_v = (cv == rv + off + 8).astype(jnp.float32)
  return _mm(p, place_p) + _mm(pv32, place_v)


def _edge_call(g, eat, Wkva, Wkvr):
  be = 3200
  nblk = ER // be

  def body(ga_ref, gb_ref, eat_a, eat_b, wkva, wkvr, rows_ref):
    rows_ref[...] = (
        _edge_rows(ga_ref[...], eat_a[...], wkva[...], wkvr[...], 0)
        + _edge_rows(gb_ref[...], eat_b[...], wkva[...], wkvr[...], ACC_W))

  return pl.pallas_call(
      body,
      grid=(nblk,),
      in_specs=[
          pl.BlockSpec((be, GW), lambda i: (i, 0)),
          pl.BlockSpec((be, GW), lambda i: (i + nblk, 0)),
          pl.BlockSpec((D_EDGE, be), lambda i: (0, i)),
          pl.BlockSpec((D_EDGE, be), lambda i: (0, i + nblk)),
          pl.BlockSpec((D_EDGE, 2 * HID), lambda i: (0, 0)),
          pl.BlockSpec((NUM_RBF, 2 * HID), lambda i: (0, 0)),
      ],
      out_specs=pl.BlockSpec((be, GW), lambda i: (i, 0)),
      out_shape=jax.ShapeDtypeStruct((ER, GW), jnp.float32),
  )(g, g, eat, eat, Wkva, Wkvr)


def _combine1_call(part, pos, Wq, Wkt, Wvt):
  bn = 5000

  def body(part_ref, pos_ref, wq, wk, wv, q_ref, k_ref, v_ref):
    a = part_ref[...]
    den = jnp.dot(a[:, 0:4], _selt(), preferred_element_type=jnp.float32) + 1e-9
    h = jnp.maximum(a[:, 8:40] / den, 0.0)
    pb = pos_ref[...]
    pad = jnp.zeros((bn, AUG - HID - 3), jnp.float32)
    q = jnp.dot(h, wq[...], preferred_element_type=jnp.float32)
    k = jnp.dot(h, wk[...], preferred_element_type=jnp.float32)
    q_ref[...] = jnp.concatenate([q, pb, pad], axis=1)
    k_ref[...] = jnp.concatenate([k, pb, pad], axis=1)
    v_ref[...] = jnp.dot(h, wv[...], preferred_element_type=jnp.float32)

  return pl.pallas_call(
      body,
      grid=(N // bn,),
      in_specs=[
          pl.BlockSpec((bn, ACC_W), lambda i: (i, 0)),
          pl.BlockSpec((bn, 3), lambda i: (i, 0)),
          pl.BlockSpec((HID, HID), lambda i: (0, 0)),
          pl.BlockSpec((HID, HID), lambda i: (0, 0)),
          pl.BlockSpec((HID, HID), lambda i: (0, 0)),
      ],
      out_specs=[
          pl.BlockSpec((bn, AUG), lambda i: (i, 0)),
          pl.BlockSpec((bn, AUG), lambda i: (i, 0)),
          pl.BlockSpec((bn, HID), lambda i: (i, 0)),
      ],
      out_shape=[
          jax.ShapeDtypeStruct((N, AUG), jnp.float32),
          jax.ShapeDtypeStruct((N, AUG), jnp.float32),
          jax.ShapeDtypeStruct((N, HID), jnp.float32),
      ],
  )(part, pos, Wq, Wkt, Wvt)


def _combine2_call(part):
  bn = 5000

  def body(part_ref, h_ref):
    a = part_ref[...]
    den = jnp.dot(a[:, 0:4], _selt(), preferred_element_type=jnp.float32) + 1e-9
    h_ref[...] = a[:, 8:40] / den

  return pl.pallas_call(
      body,
      grid=(N // bn,),
      in_specs=[pl.BlockSpec((bn, ACC_W), lambda i: (i, 0))],
      out_specs=pl.BlockSpec((bn, HID), lambda i: (i, 0)),
      out_shape=jax.ShapeDtypeStruct((N, HID), jnp.float32),
  )(part)


def kernel(x, edge_index, edge_attr, pos, Wq1, Wk1, Wv1, Wq2, Wk2, Wv2):
  src2d = edge_index[0].reshape(E // SUB, SUB)
  dst2d = edge_index[1].reshape(E // SUB, SUB)
  eat = edge_attr.T

  Wk1t, Wk1a, Wk1r = Wk1[:D_IN], Wk1[D_IN:D_IN + D_EDGE], Wk1[D_IN + D_EDGE:]
  Wv1t, Wv1a, Wv1r = Wv1[:D_IN], Wv1[D_IN:D_IN + D_EDGE], Wv1[D_IN + D_EDGE:]
  Wk2t, Wk2a, Wk2r = Wk2[:HID], Wk2[HID:HID + D_EDGE], Wk2[HID + D_EDGE:]
  Wv2t, Wv2a, Wv2r = Wv2[:HID], Wv2[HID:HID + D_EDGE], Wv2[HID + D_EDGE:]

  zeros = jnp.zeros((ZR2, ACC_W), jnp.float32)
  dstc2d = _dstidx_call(edge_index[1]).reshape(NUM_CORES, E // SUB, SUB)

  Wkv1a = jnp.concatenate([Wk1a, Wv1a], axis=1)
  Wkv1r = jnp.concatenate([Wk1r, Wv1r], axis=1)
  Wkv2a = jnp.concatenate([Wk2a, Wv2a], axis=1)
  Wkv2r = jnp.concatenate([Wk2r, Wv2r], axis=1)

  tq1, tk1, tv1 = _prep_call(x, pos, Wq1, Wk1t, Wv1t)
  g1 = _gather_call(src2d, dst2d, tq1, tk1, tv1)
  rows1 = _edge_call(g1, eat, Wkv1a, Wkv1r)
  part1 = _scatter_call(dstc2d, rows1, zeros)

  tq2, tk2, tv2 = _combine1_call(part1, pos, Wq2, Wk2t, Wv2t)
  g2 = _gather_call(src2d, dst2d, tq2, tk2, tv2)
  rows2 = _edge_call(g2, eat, Wkv2a, Wkv2r)
  part2 = _scatter_call(dstc2d, rows2, zeros)
  return _combine2_call(part2)
